# Initial kernel scaffold; baseline (speedup 1.0000x reference)
import jax
import jax.numpy as jnp
from jax.experimental import pallas as pl


def kernel(x, edge_index, batch, W1, b1, W2, b2, W3, b3, CW1, Cb1, CW2, Cb2, CW3, Cb3):
    raise NotImplementedError("write your pallas kernel here")


# speedup vs baseline: 8.2658x; 8.2658x over previous
import functools

import jax
import jax.numpy as jnp
from jax import lax
from jax.experimental import pallas as pl
from jax.experimental.pallas import tpu as pltpu
from jax.experimental.pallas import tpu_sc as plsc

N = 50000
NP = 50048
H = NP // 2
HROWS = H + 8
SROWS = NP + 16
E = 800000
EP = 819200
F = 64
NG = 64
R = 6256

_MESH = plsc.VectorSubcoreMesh(
    core_axis_name="c", subcore_axis_name="s", num_cores=2, num_subcores=16
)
_SC_PARAMS = pltpu.CompilerParams(use_tc_tiling_on_sc=False)
_f32 = jnp.float32


@functools.partial(
    pl.kernel,
    out_type=jax.ShapeDtypeStruct((2 * NP, 16), _f32),
    mesh=_MESH,
    compiler_params=_SC_PARAMS,
    scratch_types=[
        pltpu.VMEM_SHARED((SROWS, 16), _f32),
        pltpu.VMEM((128,), jnp.int32),
        pltpu.VMEM((128, 16), _f32),
    ],
)
def _deg_pass(dst_hbm, ones_hbm, zeros_hbm, out_hbm, acc, didx, ones_v):
    cid = lax.axis_index("c")
    sid = lax.axis_index("s")
    wid = cid * 16 + sid
    pltpu.sync_copy(zeros_hbm.at[pl.ds(0, 3128)], acc.at[pl.ds(sid * 3128, 3128)])

    @pl.when(sid == 0)
    def _():
        pltpu.sync_copy(zeros_hbm.at[pl.ds(0, 16)], acc.at[pl.ds(NP, 16)])

    pltpu.sync_copy(ones_hbm, ones_v)
    plsc.subcore_barrier()

    def body(t, carry):
        base = wid * 25600 + t * 128
        pltpu.sync_copy(dst_hbm.at[pl.ds(base, 128)], didx)
        pltpu.sync_copy(ones_v, acc.at[didx], add=True)
        return carry

    lax.fori_loop(0, 200, body, 0)
    plsc.subcore_barrier()
    pltpu.sync_copy(
        acc.at[pl.ds(sid * 3128, 3128)],
        out_hbm.at[pl.ds(cid * NP + sid * 3128, 3128)],
    )


@functools.partial(
    pl.kernel,
    out_type=jax.ShapeDtypeStruct((2 * NP, 16), _f32),
    mesh=_MESH,
    compiler_params=_SC_PARAMS,
    scratch_types=[
        pltpu.VMEM_SHARED((SROWS, 16), _f32),
        pltpu.VMEM((128,), jnp.int32),
        pltpu.VMEM((128,), jnp.int32),
        pltpu.VMEM((128, 16), _f32),
        pltpu.SemaphoreType.DMA,
    ],
)
def _scalar_pass(t1_hbm, src_hbm, dst_hbm, zeros_hbm, out_hbm,
                 acc, sidx, didx, vals, sem):
    cid = lax.axis_index("c")
    sid = lax.axis_index("s")
    wid = cid * 16 + sid
    pltpu.sync_copy(zeros_hbm.at[pl.ds(0, 3128)], acc.at[pl.ds(sid * 3128, 3128)])

    @pl.when(sid == 0)
    def _():
        pltpu.sync_copy(zeros_hbm.at[pl.ds(0, 16)], acc.at[pl.ds(NP, 16)])

    plsc.subcore_barrier()

    def body(t, carry):
        base = wid * 25600 + t * 128
        pltpu.sync_copy(src_hbm.at[pl.ds(base, 128)], sidx)
        pltpu.sync_copy(dst_hbm.at[pl.ds(base, 128)], didx)
        pltpu.async_copy(t1_hbm.at[sidx], vals, sem).wait()
        pltpu.sync_copy(vals, acc.at[didx], add=True)
        return carry

    lax.fori_loop(0, 200, body, 0)
    plsc.subcore_barrier()
    pltpu.sync_copy(
        acc.at[pl.ds(sid * 3128, 3128)],
        out_hbm.at[pl.ds(cid * NP + sid * 3128, 3128)],
    )


@functools.partial(
    pl.kernel,
    out_type=jax.ShapeDtypeStruct((2 * H, F), _f32),
    mesh=_MESH,
    compiler_params=_SC_PARAMS,
    scratch_types=[
        pltpu.VMEM_SHARED((HROWS, F), _f32),
        pltpu.VMEM((128,), jnp.int32),
        pltpu.VMEM((128,), jnp.int32),
        pltpu.VMEM((128,), jnp.int32),
        pltpu.VMEM((128, F), _f32),
        pltpu.SemaphoreType.DMA,
    ],
)
def _edge_pass(g_hbm, src_hbm, dst_hbm, zrow_hbm, out_hbm,
               acc, sidx, didx, dloc, rows, sem):
    cid = lax.axis_index("c")
    sid = lax.axis_index("s")
    base_node = cid * H
    pltpu.sync_copy(zrow_hbm.at[pl.ds(0, 1564)], acc.at[pl.ds(sid * 1564, 1564)])

    @pl.when(sid == 0)
    def _():
        pltpu.sync_copy(zrow_hbm.at[pl.ds(0, 8)], acc.at[pl.ds(H, 8)])

    plsc.subcore_barrier()

    def body(t, carry):
        base = sid * 51200 + t * 128
        pltpu.sync_copy(src_hbm.at[pl.ds(base, 128)], sidx)
        pltpu.sync_copy(dst_hbm.at[pl.ds(base, 128)], didx)
        for j in range(8):
            d = didx[pl.ds(j * 16, 16)]
            loc = d - base_node
            ok = (loc >= 0) & (loc < H)
            dloc[pl.ds(j * 16, 16)] = jnp.where(ok, loc, H)
        pltpu.async_copy(g_hbm.at[sidx], rows, sem).wait()
        pltpu.sync_copy(rows, acc.at[dloc], add=True)
        return carry

    lax.fori_loop(0, 400, body, 0)
    plsc.subcore_barrier()
    pltpu.sync_copy(
        acc.at[pl.ds(sid * 1564, 1564)],
        out_hbm.at[pl.ds(cid * H + sid * 1564, 1564)],
    )


def _tc_prep_body(degp_ref, x_ref, dinv_ref, t1_ref):
    deg = degp_ref[0] + degp_ref[1] + 1.0
    idx = (
        lax.broadcasted_iota(jnp.int32, (391, 128), 0) * 128
        + lax.broadcasted_iota(jnp.int32, (391, 128), 1)
    )
    dinv = jnp.where(idx < N, lax.rsqrt(deg), 0.0)
    dinv_ref[...] = dinv
    t1_ref[...] = dinv * x_ref[...]


def _tc_q_body(sp_ref, t1_ref, dinv_ref, q_ref):
    q_ref[...] = dinv_ref[...] * (sp_ref[0] + sp_ref[1] + t1_ref[...])


def _tc_l1_body(q_ref, dinv_ref, w1_ref, b1_ref, w2_ref, g2_ref):
    h1 = jax.nn.relu(q_ref[...] * w1_ref[...] + b1_ref[...])
    g2_ref[...] = dinv_ref[...] * jnp.dot(
        h1, w2_ref[...], preferred_element_type=_f32
    )


def _tc_layer_body(s_ref, g_ref, dinv_ref, b_ref, w_ref, out_ref):
    h = jax.nn.relu(dinv_ref[...] * (s_ref[...] + g_ref[...]) + b_ref[...])
    out_ref[...] = dinv_ref[...] * jnp.dot(
        h, w_ref[...], preferred_element_type=_f32
    )


def _tc_final_body(s_ref, g_ref, dinv_ref, b3_ref, batch_ref,
                   cw1_ref, cb1_ref, cw2_ref, cb2_ref, cw3_ref, cb3_ref,
                   z_ref, psum, cnt):
    i = pl.program_id(0)

    @pl.when(i == 0)
    def _():
        psum[...] = jnp.zeros((NG, F), _f32)
        cnt[...] = jnp.zeros((NG, F), _f32)

    h3 = jax.nn.relu(
        dinv_ref[...] * (s_ref[...] + g_ref[...]) + b3_ref[...]
    )
    seg = lax.broadcasted_iota(jnp.int32, (R, NG), 1)
    mf = (batch_ref[...] == seg).astype(_f32)
    dn = (((0,), (0,)), ((), ()))
    psum[...] += lax.dot_general(mf, h3, dn, preferred_element_type=_f32)
    cnt[...] += lax.dot_general(
        mf, jnp.ones((R, F), _f32), dn, preferred_element_type=_f32
    )

    @pl.when(i == pl.num_programs(0) - 1)
    def _():
        pooled = psum[...] / jnp.maximum(cnt[...], 1.0)
        z = jax.nn.relu(
            jnp.dot(pooled, cw1_ref[...], preferred_element_type=_f32)
            + cb1_ref[...]
        )
        z = jax.nn.relu(
            jnp.dot(z, cw2_ref[...], preferred_element_type=_f32)
            + cb2_ref[...]
        )
        z_ref[...] = (
            jnp.dot(z, cw3_ref[...], preferred_element_type=_f32)
            + cb3_ref[...]
        )


def _whole(shape):
    return pl.BlockSpec(shape, lambda i: tuple(0 for _ in shape))


def _rows(shape):
    return pl.BlockSpec(shape, lambda i: (i,) + tuple(0 for _ in shape[1:]))


def kernel(x, edge_index, batch, W1, b1, W2, b2, W3, b3,
           CW1, Cb1, CW2, Cb2, CW3, Cb3):
    ei = edge_index.astype(jnp.int32)
    src = jnp.concatenate([ei[0], jnp.zeros((EP - E,), jnp.int32)])
    dst = jnp.concatenate([ei[1], jnp.full((EP - E,), NP, jnp.int32)])
    xp = jnp.pad(x[:, 0], (0, NP - N)).reshape(391, 128)
    ones_col = jnp.ones((128, 16), _f32)
    zeros_col = jnp.zeros((3128, 16), _f32)
    zrow = jnp.zeros((1564, F), _f32)

    degp = _deg_pass(dst, ones_col, zeros_col)[:, 0].reshape(2, 391, 128)
    dinv2, t12 = pl.pallas_call(
        _tc_prep_body,
        out_shape=(
            jax.ShapeDtypeStruct((391, 128), _f32),
            jax.ShapeDtypeStruct((391, 128), _f32),
        ),
    )(degp, xp)

    t1w = jnp.pad(t12.reshape(NP, 1), ((0, 0), (0, 15)))
    sp = _scalar_pass(t1w, src, dst, zeros_col)
    q2 = pl.pallas_call(
        _tc_q_body,
        out_shape=jax.ShapeDtypeStruct((391, 128), _f32),
    )(sp[:, 0].reshape(2, 391, 128), t12, dinv2)

    dinv_col = dinv2.reshape(NP, 1)
    g2 = pl.pallas_call(
        _tc_l1_body,
        grid=(NP // R,),
        in_specs=[
            _rows((R, 1)),
            _rows((R, 1)),
            _whole((1, F)),
            _whole((1, F)),
            _whole((F, F)),
        ],
        out_specs=_rows((R, F)),
        out_shape=jax.ShapeDtypeStruct((NP, F), _f32),
    )(q2.reshape(NP, 1), dinv_col, W1, b1.reshape(1, F), W2)

    layer = pl.pallas_call(
        _tc_layer_body,
        grid=(NP // R,),
        in_specs=[
            _rows((R, F)),
            _rows((R, F)),
            _rows((R, 1)),
            _whole((1, F)),
            _whole((F, F)),
        ],
        out_specs=_rows((R, F)),
        out_shape=jax.ShapeDtypeStruct((NP, F), _f32),
    )

    s2 = _edge_pass(g2, src, dst, zrow).reshape(NP, F)
    g3 = layer(s2, g2, dinv_col, b2.reshape(1, F), W3)
    s3 = _edge_pass(g3, src, dst, zrow).reshape(NP, F)

    bp = jnp.concatenate(
        [batch.astype(jnp.int32), jnp.full((NP - N,), 1 << 20, jnp.int32)]
    ).reshape(NP, 1)
    z = pl.pallas_call(
        _tc_final_body,
        grid=(NP // R,),
        in_specs=[
            _rows((R, F)),
            _rows((R, F)),
            _rows((R, 1)),
            _whole((1, F)),
            _rows((R, 1)),
            _whole((F, F // 2)),
            _whole((1, F // 2)),
            _whole((F // 2, F // 4)),
            _whole((1, F // 4)),
            _whole((F // 4, 4)),
            _whole((1, 4)),
        ],
        out_specs=_whole((NG, 4)),
        out_shape=jax.ShapeDtypeStruct((NG, 4), _f32),
        scratch_shapes=[
            pltpu.VMEM((NG, F), _f32),
            pltpu.VMEM((NG, F), _f32),
        ],
    )(s3, g3, dinv_col, b3.reshape(1, F), bp,
      CW1, Cb1.reshape(1, F // 2), CW2, Cb2.reshape(1, F // 4),
      CW3, Cb3.reshape(1, 4))
    return z


# speedup vs baseline: 9.4438x; 1.1425x over previous
import functools

import jax
import jax.numpy as jnp
from jax import lax
from jax.experimental import pallas as pl
from jax.experimental.pallas import tpu as pltpu
from jax.experimental.pallas import tpu_sc as plsc

N = 50000
NP = 50048
H = NP // 2
HROWS = H + 8
SROWS = NP + 16
E = 800000
EP = 819200
F = 64
NG = 64
R = 6256

_MESH = plsc.VectorSubcoreMesh(
    core_axis_name="c", subcore_axis_name="s", num_cores=2, num_subcores=16
)
_SC_PARAMS = pltpu.CompilerParams(use_tc_tiling_on_sc=False)
_f32 = jnp.float32


@functools.partial(
    pl.kernel,
    out_type=jax.ShapeDtypeStruct((2 * NP, 16), _f32),
    mesh=_MESH,
    compiler_params=_SC_PARAMS,
    scratch_types=[
        pltpu.VMEM_SHARED((SROWS, 16), _f32),
        pltpu.VMEM((128,), jnp.int32),
        pltpu.VMEM((128, 16), _f32),
    ],
)
def _deg_pass(dst_hbm, ones_hbm, zeros_hbm, out_hbm, acc, didx, ones_v):
    cid = lax.axis_index("c")
    sid = lax.axis_index("s")
    wid = cid * 16 + sid
    pltpu.sync_copy(zeros_hbm.at[pl.ds(0, 3128)], acc.at[pl.ds(sid * 3128, 3128)])

    @pl.when(sid == 0)
    def _():
        pltpu.sync_copy(zeros_hbm.at[pl.ds(0, 16)], acc.at[pl.ds(NP, 16)])

    pltpu.sync_copy(ones_hbm, ones_v)
    plsc.subcore_barrier()

    def body(t, carry):
        base = wid * 25600 + t * 128
        pltpu.sync_copy(dst_hbm.at[pl.ds(base, 128)], didx)
        pltpu.sync_copy(ones_v, acc.at[didx], add=True)
        return carry

    lax.fori_loop(0, 200, body, 0)
    plsc.subcore_barrier()
    pltpu.sync_copy(
        acc.at[pl.ds(sid * 3128, 3128)],
        out_hbm.at[pl.ds(cid * NP + sid * 3128, 3128)],
    )


@functools.partial(
    pl.kernel,
    out_type=jax.ShapeDtypeStruct((2 * NP, 16), _f32),
    mesh=_MESH,
    compiler_params=_SC_PARAMS,
    scratch_types=[
        pltpu.VMEM_SHARED((SROWS, 16), _f32),
        pltpu.VMEM((128,), jnp.int32),
        pltpu.VMEM((128,), jnp.int32),
        pltpu.VMEM((128, 16), _f32),
        pltpu.SemaphoreType.DMA,
    ],
)
def _scalar_pass(t1_hbm, src_hbm, dst_hbm, zeros_hbm, out_hbm,
                 acc, sidx, didx, vals, sem):
    cid = lax.axis_index("c")
    sid = lax.axis_index("s")
    wid = cid * 16 + sid
    pltpu.sync_copy(zeros_hbm.at[pl.ds(0, 3128)], acc.at[pl.ds(sid * 3128, 3128)])

    @pl.when(sid == 0)
    def _():
        pltpu.sync_copy(zeros_hbm.at[pl.ds(0, 16)], acc.at[pl.ds(NP, 16)])

    plsc.subcore_barrier()

    def body(t, carry):
        base = wid * 25600 + t * 128
        pltpu.sync_copy(src_hbm.at[pl.ds(base, 128)], sidx)
        pltpu.sync_copy(dst_hbm.at[pl.ds(base, 128)], didx)
        pltpu.async_copy(t1_hbm.at[sidx], vals, sem).wait()
        pltpu.sync_copy(vals, acc.at[didx], add=True)
        return carry

    lax.fori_loop(0, 200, body, 0)
    plsc.subcore_barrier()
    pltpu.sync_copy(
        acc.at[pl.ds(sid * 3128, 3128)],
        out_hbm.at[pl.ds(cid * NP + sid * 3128, 3128)],
    )


_C = 64
_NB = 4
_NCH = 51200 // _C


@functools.partial(
    pl.kernel,
    out_type=jax.ShapeDtypeStruct((2 * H, F), _f32),
    mesh=_MESH,
    compiler_params=_SC_PARAMS,
    scratch_types=[
        pltpu.VMEM_SHARED((HROWS, F), _f32),
        pltpu.VMEM((_NB, _C), jnp.int32),
        pltpu.VMEM((_NB, _C), jnp.int32),
        pltpu.VMEM((_NB, _C), jnp.int32),
        pltpu.VMEM((_NB, _C, F), _f32),
    ]
    + [pltpu.SemaphoreType.DMA] * 12,
)
def _edge_pass(g_hbm, src_hbm, dst_hbm, zrow_hbm, out_hbm,
               acc, sidx, didx, dloc, rows, *sems):
    cid = lax.axis_index("c")
    sid = lax.axis_index("s")
    base_node = cid * H
    isem = sems[0:4]
    gsem = sems[4:8]
    ssem = sems[8:12]
    pltpu.sync_copy(zrow_hbm.at[pl.ds(0, 1564)], acc.at[pl.ds(sid * 1564, 1564)])

    @pl.when(sid == 0)
    def _():
        pltpu.sync_copy(zrow_hbm.at[pl.ds(0, 8)], acc.at[pl.ds(H, 8)])

    plsc.subcore_barrier()

    def idx_issue(t, b):
        base = sid * 51200 + t * _C
        pltpu.async_copy(src_hbm.at[pl.ds(base, _C)], sidx.at[b], isem[b])
        pltpu.async_copy(dst_hbm.at[pl.ds(base, _C)], didx.at[b], isem[b])

    def idx_wait(t, b):
        base = sid * 51200 + t * _C
        pltpu.make_async_copy(src_hbm.at[pl.ds(base, _C)], sidx.at[b],
                              isem[b]).wait()
        pltpu.make_async_copy(dst_hbm.at[pl.ds(base, _C)], didx.at[b],
                              isem[b]).wait()

    def remap(b):
        for j in range(_C // 16):
            d = didx[b, pl.ds(j * 16, 16)]
            loc = d - base_node
            ok = (loc >= 0) & (loc < H)
            dloc[b, pl.ds(j * 16, 16)] = jnp.where(ok, loc, H)

    def scat_drain(b):
        pltpu.make_async_copy(rows.at[b], acc.at[dloc.at[b]], ssem[b]).wait()

    for t0 in range(3):
        idx_issue(t0, t0)

    def step(h, k):
        t = h * 4 + k
        b = k
        a = (k - 1) % 4

        def drain():
            scat_drain(b)

        def gwait_and_scat():
            pltpu.make_async_copy(g_hbm.at[sidx.at[a]], rows.at[a],
                                  gsem[a]).wait()
            pltpu.async_copy(rows.at[a], acc.at[dloc.at[a]], ssem[a],
                             add=True)

        def prefetch():
            idx_issue(t + 3, a)

        pl.when(h >= 1)(drain)
        idx_wait(t, b)
        remap(b)
        pltpu.async_copy(g_hbm.at[sidx.at[b]], rows.at[b], gsem[b])
        if k == 0:
            pl.when(h >= 1)(gwait_and_scat)
            prefetch()
        else:
            gwait_and_scat()
            pl.when(h <= 198)(prefetch)

    def body(h, carry):
        for k in range(4):
            step(h, k)
        return carry

    lax.fori_loop(0, _NCH // 4, body, 0)
    pltpu.make_async_copy(g_hbm.at[sidx.at[3]], rows.at[3], gsem[3]).wait()
    pltpu.async_copy(rows.at[3], acc.at[dloc.at[3]], ssem[3], add=True)
    for b in range(4):
        scat_drain(b)
    plsc.subcore_barrier()
    pltpu.sync_copy(
        acc.at[pl.ds(sid * 1564, 1564)],
        out_hbm.at[pl.ds(cid * H + sid * 1564, 1564)],
    )


def _tc_prep_body(degp_ref, x_ref, dinv_ref, t1_ref):
    deg = degp_ref[0] + degp_ref[1] + 1.0
    idx = (
        lax.broadcasted_iota(jnp.int32, (391, 128), 0) * 128
        + lax.broadcasted_iota(jnp.int32, (391, 128), 1)
    )
    dinv = jnp.where(idx < N, lax.rsqrt(deg), 0.0)
    dinv_ref[...] = dinv
    t1_ref[...] = dinv * x_ref[...]


def _tc_q_body(sp_ref, t1_ref, dinv_ref, q_ref):
    q_ref[...] = dinv_ref[...] * (sp_ref[0] + sp_ref[1] + t1_ref[...])


def _tc_l1_body(q_ref, dinv_ref, w1_ref, b1_ref, w2_ref, g2_ref):
    h1 = jax.nn.relu(q_ref[...] * w1_ref[...] + b1_ref[...])
    g2_ref[...] = dinv_ref[...] * jnp.dot(
        h1, w2_ref[...], preferred_element_type=_f32
    )


def _tc_layer_body(s_ref, g_ref, dinv_ref, b_ref, w_ref, out_ref):
    h = jax.nn.relu(dinv_ref[...] * (s_ref[...] + g_ref[...]) + b_ref[...])
    out_ref[...] = dinv_ref[...] * jnp.dot(
        h, w_ref[...], preferred_element_type=_f32
    )


def _tc_final_body(s_ref, g_ref, dinv_ref, b3_ref, batch_ref,
                   cw1_ref, cb1_ref, cw2_ref, cb2_ref, cw3_ref, cb3_ref,
                   z_ref, psum, cnt):
    i = pl.program_id(0)

    @pl.when(i == 0)
    def _():
        psum[...] = jnp.zeros((NG, F), _f32)
        cnt[...] = jnp.zeros((NG, F), _f32)

    h3 = jax.nn.relu(
        dinv_ref[...] * (s_ref[...] + g_ref[...]) + b3_ref[...]
    )
    seg = lax.broadcasted_iota(jnp.int32, (R, NG), 1)
    mf = (batch_ref[...] == seg).astype(_f32)
    dn = (((0,), (0,)), ((), ()))
    psum[...] += lax.dot_general(mf, h3, dn, preferred_element_type=_f32)
    cnt[...] += lax.dot_general(
        mf, jnp.ones((R, F), _f32), dn, preferred_element_type=_f32
    )

    @pl.when(i == pl.num_programs(0) - 1)
    def _():
        pooled = psum[...] / jnp.maximum(cnt[...], 1.0)
        z = jax.nn.relu(
            jnp.dot(pooled, cw1_ref[...], preferred_element_type=_f32)
            + cb1_ref[...]
        )
        z = jax.nn.relu(
            jnp.dot(z, cw2_ref[...], preferred_element_type=_f32)
            + cb2_ref[...]
        )
        z_ref[...] = (
            jnp.dot(z, cw3_ref[...], preferred_element_type=_f32)
            + cb3_ref[...]
        )


def _whole(shape):
    return pl.BlockSpec(shape, lambda i: tuple(0 for _ in shape))


def _rows(shape):
    return pl.BlockSpec(shape, lambda i: (i,) + tuple(0 for _ in shape[1:]))


def kernel(x, edge_index, batch, W1, b1, W2, b2, W3, b3,
           CW1, Cb1, CW2, Cb2, CW3, Cb3):
    ei = edge_index.astype(jnp.int32)
    src = jnp.concatenate([ei[0], jnp.zeros((EP - E,), jnp.int32)])
    dst = jnp.concatenate([ei[1], jnp.full((EP - E,), NP, jnp.int32)])
    xp = jnp.pad(x[:, 0], (0, NP - N)).reshape(391, 128)
    ones_col = jnp.ones((128, 16), _f32)
    zeros_col = jnp.zeros((3128, 16), _f32)
    zrow = jnp.zeros((1564, F), _f32)

    degp = _deg_pass(dst, ones_col, zeros_col)[:, 0].reshape(2, 391, 128)
    dinv2, t12 = pl.pallas_call(
        _tc_prep_body,
        out_shape=(
            jax.ShapeDtypeStruct((391, 128), _f32),
            jax.ShapeDtypeStruct((391, 128), _f32),
        ),
    )(degp, xp)

    t1w = jnp.pad(t12.reshape(NP, 1), ((0, 0), (0, 15)))
    sp = _scalar_pass(t1w, src, dst, zeros_col)
    q2 = pl.pallas_call(
        _tc_q_body,
        out_shape=jax.ShapeDtypeStruct((391, 128), _f32),
    )(sp[:, 0].reshape(2, 391, 128), t12, dinv2)

    dinv_col = dinv2.reshape(NP, 1)
    g2 = pl.pallas_call(
        _tc_l1_body,
        grid=(NP // R,),
        in_specs=[
            _rows((R, 1)),
            _rows((R, 1)),
            _whole((1, F)),
            _whole((1, F)),
            _whole((F, F)),
        ],
        out_specs=_rows((R, F)),
        out_shape=jax.ShapeDtypeStruct((NP, F), _f32),
    )(q2.reshape(NP, 1), dinv_col, W1, b1.reshape(1, F), W2)

    layer = pl.pallas_call(
        _tc_layer_body,
        grid=(NP // R,),
        in_specs=[
            _rows((R, F)),
            _rows((R, F)),
            _rows((R, 1)),
            _whole((1, F)),
            _whole((F, F)),
        ],
        out_specs=_rows((R, F)),
        out_shape=jax.ShapeDtypeStruct((NP, F), _f32),
    )

    s2 = _edge_pass(g2, src, dst, zrow).reshape(NP, F)
    g3 = layer(s2, g2, dinv_col, b2.reshape(1, F), W3)
    s3 = _edge_pass(g3, src, dst, zrow).reshape(NP, F)

    bp = jnp.concatenate(
        [batch.astype(jnp.int32), jnp.full((NP - N,), 1 << 20, jnp.int32)]
    ).reshape(NP, 1)
    z = pl.pallas_call(
        _tc_final_body,
        grid=(NP // R,),
        in_specs=[
            _rows((R, F)),
            _rows((R, F)),
            _rows((R, 1)),
            _whole((1, F)),
            _rows((R, 1)),
            _whole((F, F // 2)),
            _whole((1, F // 2)),
            _whole((F // 2, F // 4)),
            _whole((1, F // 4)),
            _whole((F // 4, 4)),
            _whole((1, 4)),
        ],
        out_specs=_whole((NG, 4)),
        out_shape=jax.ShapeDtypeStruct((NG, 4), _f32),
        scratch_shapes=[
            pltpu.VMEM((NG, F), _f32),
            pltpu.VMEM((NG, F), _f32),
        ],
    )(s3, g3, dinv_col, b3.reshape(1, F), bp,
      CW1, Cb1.reshape(1, F // 2), CW2, Cb2.reshape(1, F // 4),
      CW3, Cb3.reshape(1, 4))
    return z


# speedup vs baseline: 9.6307x; 1.0198x over previous
import functools

import jax
import jax.numpy as jnp
from jax import lax
from jax.experimental import pallas as pl
from jax.experimental.pallas import tpu as pltpu
from jax.experimental.pallas import tpu_sc as plsc

N = 50000
NP = 50048
H = NP // 2
HROWS = H + 8
SROWS = NP + 16
E = 800000
EP = 823296
F = 64
NG = 64
R = 6256

_MESH = plsc.VectorSubcoreMesh(
    core_axis_name="c", subcore_axis_name="s", num_cores=2, num_subcores=16
)
_SC_PARAMS = pltpu.CompilerParams(use_tc_tiling_on_sc=False)
_f32 = jnp.float32


_NB = 3


def _sc_pipeline(nch, idx_issue, idx_wait, prep, gather_issue, gather_wait,
                 scat_issue, scat_drain):
    for t0 in range(2):
        idx_issue(t0, t0)

    hmax = {1: (nch - 4) // 3, 2: (nch - 5) // 3}

    def step(h, k):
        t = h * 3 + k
        b = k
        a = (k - 1) % 3

        def drain():
            scat_drain(b)

        def gwait_scat():
            gather_wait(a)
            scat_issue(a)

        def prefetch():
            idx_issue(t + 2, a)

        pl.when(h >= 1)(drain)
        idx_wait(t, b)
        prep(b)
        gather_issue(t, b)
        if k == 0:
            pl.when(h >= 1)(gwait_scat)
            prefetch()
        else:
            gwait_scat()
            pl.when(h <= hmax[k])(prefetch)

    def body(h, carry):
        for k in range(3):
            step(h, k)
        return carry

    lax.fori_loop(0, nch // 3, body, 0)
    last = (nch - 1) % 3
    gather_wait(last)
    scat_issue(last)
    for b in range(3):
        scat_drain(b)


@functools.partial(
    pl.kernel,
    out_type=jax.ShapeDtypeStruct((2 * NP, 16), _f32),
    mesh=_MESH,
    compiler_params=_SC_PARAMS,
    scratch_types=[
        pltpu.VMEM_SHARED((SROWS, 16), _f32),
        pltpu.VMEM((_NB, 2, 128), jnp.int32),
        pltpu.VMEM((_NB, 128), jnp.int32),
        pltpu.VMEM((128, 16), _f32),
    ]
    + [pltpu.SemaphoreType.DMA] * 6,
)
def _deg_pass(eic_hbm, ones_hbm, zeros_hbm, out_hbm, acc, idxb, dwr, ones_v,
              *sems):
    cid = lax.axis_index("c")
    sid = lax.axis_index("s")
    wid = cid * 16 + sid
    isem = sems[0:3]
    ssem = sems[3:6]
    pltpu.sync_copy(zeros_hbm.at[pl.ds(0, 3128)], acc.at[pl.ds(sid * 3128, 3128)])

    @pl.when(sid == 0)
    def _():
        pltpu.sync_copy(zeros_hbm.at[pl.ds(0, 16)], acc.at[pl.ds(NP, 16)])

    pltpu.sync_copy(ones_hbm, ones_v)
    plsc.subcore_barrier()

    def idx_issue(t, b):
        pltpu.async_copy(eic_hbm.at[wid * 201 + t], idxb.at[b], isem[b])

    def idx_wait(t, b):
        pltpu.make_async_copy(eic_hbm.at[wid * 201 + t], idxb.at[b],
                              isem[b]).wait()

    def prep(b):
        for j in range(8):
            dwr[b, pl.ds(j * 16, 16)] = idxb[b, 1, pl.ds(j * 16, 16)]

    def scat_issue(b):
        pltpu.async_copy(ones_v, acc.at[dwr.at[b]], ssem[b], add=True)

    def scat_drain(b):
        pltpu.make_async_copy(ones_v, acc.at[dwr.at[b]], ssem[b]).wait()

    _sc_pipeline(201, idx_issue, idx_wait, prep, lambda t, b: None,
                 lambda b: None, scat_issue, scat_drain)
    plsc.subcore_barrier()
    pltpu.sync_copy(
        acc.at[pl.ds(sid * 3128, 3128)],
        out_hbm.at[pl.ds(cid * NP + sid * 3128, 3128)],
    )


@functools.partial(
    pl.kernel,
    out_type=jax.ShapeDtypeStruct((2 * NP, 16), _f32),
    mesh=_MESH,
    compiler_params=_SC_PARAMS,
    scratch_types=[
        pltpu.VMEM_SHARED((SROWS, 16), _f32),
        pltpu.VMEM((_NB, 2, 128), jnp.int32),
        pltpu.VMEM((_NB, 128), jnp.int32),
        pltpu.VMEM((_NB, 128, 16), _f32),
    ]
    + [pltpu.SemaphoreType.DMA] * 9,
)
def _scalar_pass(t1_hbm, eic_hbm, zeros_hbm, out_hbm, acc, idxb, dwr, rows,
                 *sems):
    cid = lax.axis_index("c")
    sid = lax.axis_index("s")
    wid = cid * 16 + sid
    isem = sems[0:3]
    gsem = sems[3:6]
    ssem = sems[6:9]
    pltpu.sync_copy(zeros_hbm.at[pl.ds(0, 3128)], acc.at[pl.ds(sid * 3128, 3128)])

    @pl.when(sid == 0)
    def _():
        pltpu.sync_copy(zeros_hbm.at[pl.ds(0, 16)], acc.at[pl.ds(NP, 16)])

    plsc.subcore_barrier()

    def idx_issue(t, b):
        pltpu.async_copy(eic_hbm.at[wid * 201 + t], idxb.at[b], isem[b])

    def idx_wait(t, b):
        pltpu.make_async_copy(eic_hbm.at[wid * 201 + t], idxb.at[b],
                              isem[b]).wait()

    def prep(b):
        for j in range(8):
            dwr[b, pl.ds(j * 16, 16)] = idxb[b, 1, pl.ds(j * 16, 16)]

    def gather_issue(t, b):
        pltpu.async_copy(t1_hbm.at[idxb.at[b, 0]], rows.at[b], gsem[b])

    def gather_wait(b):
        pltpu.make_async_copy(t1_hbm.at[idxb.at[b, 0]], rows.at[b],
                              gsem[b]).wait()

    def scat_issue(b):
        pltpu.async_copy(rows.at[b], acc.at[dwr.at[b]], ssem[b], add=True)

    def scat_drain(b):
        pltpu.make_async_copy(rows.at[b], acc.at[dwr.at[b]], ssem[b]).wait()

    _sc_pipeline(201, idx_issue, idx_wait, prep, gather_issue, gather_wait,
                 scat_issue, scat_drain)
    plsc.subcore_barrier()
    pltpu.sync_copy(
        acc.at[pl.ds(sid * 3128, 3128)],
        out_hbm.at[pl.ds(cid * NP + sid * 3128, 3128)],
    )


@functools.partial(
    pl.kernel,
    out_type=jax.ShapeDtypeStruct((2 * H, F), _f32),
    mesh=_MESH,
    compiler_params=_SC_PARAMS,
    scratch_types=[
        pltpu.VMEM_SHARED((HROWS, F), _f32),
        pltpu.VMEM((_NB, 2, 128), jnp.int32),
        pltpu.VMEM((_NB, 128), jnp.int32),
        pltpu.VMEM((_NB, 128, F), _f32),
    ]
    + [pltpu.SemaphoreType.DMA] * 9,
)
def _edge_pass(g_hbm, eic_hbm, zrow_hbm, out_hbm, acc, idxb, dloc, rows,
               *sems):
    cid = lax.axis_index("c")
    sid = lax.axis_index("s")
    base_node = cid * H
    isem = sems[0:3]
    gsem = sems[3:6]
    ssem = sems[6:9]
    pltpu.sync_copy(zrow_hbm.at[pl.ds(0, 1564)], acc.at[pl.ds(sid * 1564, 1564)])

    @pl.when(sid == 0)
    def _():
        pltpu.sync_copy(zrow_hbm.at[pl.ds(0, 8)], acc.at[pl.ds(H, 8)])

    plsc.subcore_barrier()

    def idx_issue(t, b):
        pltpu.async_copy(eic_hbm.at[sid * 402 + t], idxb.at[b], isem[b])

    def idx_wait(t, b):
        pltpu.make_async_copy(eic_hbm.at[sid * 402 + t], idxb.at[b],
                              isem[b]).wait()

    def prep(b):
        for j in range(8):
            d = idxb[b, 1, pl.ds(j * 16, 16)]
            loc = d - base_node
            ok = (loc >= 0) & (loc < H)
            dloc[b, pl.ds(j * 16, 16)] = jnp.where(ok, loc, H)

    def gather_issue(t, b):
        pltpu.async_copy(g_hbm.at[idxb.at[b, 0]], rows.at[b], gsem[b])

    def gather_wait(b):
        pltpu.make_async_copy(g_hbm.at[idxb.at[b, 0]], rows.at[b],
                              gsem[b]).wait()

    def scat_issue(b):
        pltpu.async_copy(rows.at[b], acc.at[dloc.at[b]], ssem[b], add=True)

    def scat_drain(b):
        pltpu.make_async_copy(rows.at[b], acc.at[dloc.at[b]], ssem[b]).wait()

    _sc_pipeline(402, idx_issue, idx_wait, prep, gather_issue, gather_wait,
                 scat_issue, scat_drain)
    plsc.subcore_barrier()
    pltpu.sync_copy(
        acc.at[pl.ds(sid * 1564, 1564)],
        out_hbm.at[pl.ds(cid * H + sid * 1564, 1564)],
    )


def _tc_prep_body(degp_ref, x_ref, dinv_ref, t1_ref):
    deg = degp_ref[0] + degp_ref[1] + 1.0
    idx = (
        lax.broadcasted_iota(jnp.int32, (391, 128), 0) * 128
        + lax.broadcasted_iota(jnp.int32, (391, 128), 1)
    )
    dinv = jnp.where(idx < N, lax.rsqrt(deg), 0.0)
    dinv_ref[...] = dinv
    t1_ref[...] = dinv * x_ref[...]


def _tc_q_body(sp_ref, t1_ref, dinv_ref, q_ref):
    q_ref[...] = dinv_ref[...] * (sp_ref[0] + sp_ref[1] + t1_ref[...])


def _tc_l1_body(q_ref, dinv_ref, w1_ref, b1_ref, w2_ref, g2_ref):
    h1 = jax.nn.relu(q_ref[...] * w1_ref[...] + b1_ref[...])
    g2_ref[...] = dinv_ref[...] * jnp.dot(
        h1, w2_ref[...], preferred_element_type=_f32
    )


def _tc_layer_body(s_ref, g_ref, dinv_ref, b_ref, w_ref, out_ref):
    h = jax.nn.relu(dinv_ref[...] * (s_ref[...] + g_ref[...]) + b_ref[...])
    out_ref[...] = dinv_ref[...] * jnp.dot(
        h, w_ref[...], preferred_element_type=_f32
    )


def _tc_final_body(s_ref, g_ref, dinv_ref, b3_ref, batch_ref,
                   cw1_ref, cb1_ref, cw2_ref, cb2_ref, cw3_ref, cb3_ref,
                   z_ref, psum, cnt):
    i = pl.program_id(0)

    @pl.when(i == 0)
    def _():
        psum[...] = jnp.zeros((NG, F), _f32)
        cnt[...] = jnp.zeros((NG, F), _f32)

    h3 = jax.nn.relu(
        dinv_ref[...] * (s_ref[...] + g_ref[...]) + b3_ref[...]
    )
    seg = lax.broadcasted_iota(jnp.int32, (R, NG), 1)
    mf = (batch_ref[...] == seg).astype(_f32)
    dn = (((0,), (0,)), ((), ()))
    psum[...] += lax.dot_general(mf, h3, dn, preferred_element_type=_f32)
    cnt[...] += lax.dot_general(
        mf, jnp.ones((R, F), _f32), dn, preferred_element_type=_f32
    )

    @pl.when(i == pl.num_programs(0) - 1)
    def _():
        pooled = psum[...] / jnp.maximum(cnt[...], 1.0)
        z = jax.nn.relu(
            jnp.dot(pooled, cw1_ref[...], preferred_element_type=_f32)
            + cb1_ref[...]
        )
        z = jax.nn.relu(
            jnp.dot(z, cw2_ref[...], preferred_element_type=_f32)
            + cb2_ref[...]
        )
        z_ref[...] = (
            jnp.dot(z, cw3_ref[...], preferred_element_type=_f32)
            + cb3_ref[...]
        )


def _whole(shape):
    return pl.BlockSpec(shape, lambda i: tuple(0 for _ in shape))


def _rows(shape):
    return pl.BlockSpec(shape, lambda i: (i,) + tuple(0 for _ in shape[1:]))


def kernel(x, edge_index, batch, W1, b1, W2, b2, W3, b3,
           CW1, Cb1, CW2, Cb2, CW3, Cb3):
    ei = edge_index.astype(jnp.int32)
    src = jnp.concatenate([ei[0], jnp.zeros((EP - E,), jnp.int32)])
    dst = jnp.concatenate([ei[1], jnp.full((EP - E,), NP, jnp.int32)])
    eic = jnp.stack([src.reshape(-1, 128), dst.reshape(-1, 128)], axis=1)
    xp = jnp.pad(x[:, 0], (0, NP - N)).reshape(391, 128)
    ones_col = jnp.ones((128, 16), _f32)
    zeros_col = jnp.zeros((3128, 16), _f32)
    zrow = jnp.zeros((1564, F), _f32)

    degp = _deg_pass(eic, ones_col, zeros_col)[:, 0].reshape(2, 391, 128)
    dinv2, t12 = pl.pallas_call(
        _tc_prep_body,
        out_shape=(
            jax.ShapeDtypeStruct((391, 128), _f32),
            jax.ShapeDtypeStruct((391, 128), _f32),
        ),
    )(degp, xp)

    t1w = jnp.pad(t12.reshape(NP, 1), ((0, 0), (0, 15)))
    sp = _scalar_pass(t1w, eic, zeros_col)
    q2 = pl.pallas_call(
        _tc_q_body,
        out_shape=jax.ShapeDtypeStruct((391, 128), _f32),
    )(sp[:, 0].reshape(2, 391, 128), t12, dinv2)

    dinv_col = dinv2.reshape(NP, 1)
    g2 = pl.pallas_call(
        _tc_l1_body,
        grid=(NP // R,),
        in_specs=[
            _rows((R, 1)),
            _rows((R, 1)),
            _whole((1, F)),
            _whole((1, F)),
            _whole((F, F)),
        ],
        out_specs=_rows((R, F)),
        out_shape=jax.ShapeDtypeStruct((NP, F), _f32),
    )(q2.reshape(NP, 1), dinv_col, W1, b1.reshape(1, F), W2)

    layer = pl.pallas_call(
        _tc_layer_body,
        grid=(NP // R,),
        in_specs=[
            _rows((R, F)),
            _rows((R, F)),
            _rows((R, 1)),
            _whole((1, F)),
            _whole((F, F)),
        ],
        out_specs=_rows((R, F)),
        out_shape=jax.ShapeDtypeStruct((NP, F), _f32),
    )

    s2 = _edge_pass(g2, eic, zrow).reshape(NP, F)
    g3 = layer(s2, g2, dinv_col, b2.reshape(1, F), W3)
    s3 = _edge_pass(g3, eic, zrow).reshape(NP, F)

    bp = jnp.concatenate(
        [batch.astype(jnp.int32), jnp.full((NP - N,), 1 << 20, jnp.int32)]
    ).reshape(NP, 1)
    z = pl.pallas_call(
        _tc_final_body,
        grid=(NP // R,),
        in_specs=[
            _rows((R, F)),
            _rows((R, F)),
            _rows((R, 1)),
            _whole((1, F)),
            _rows((R, 1)),
            _whole((F, F // 2)),
            _whole((1, F // 2)),
            _whole((F // 2, F // 4)),
            _whole((1, F // 4)),
            _whole((F // 4, 4)),
            _whole((1, 4)),
        ],
        out_specs=_whole((NG, 4)),
        out_shape=jax.ShapeDtypeStruct((NG, 4), _f32),
        scratch_shapes=[
            pltpu.VMEM((NG, F), _f32),
            pltpu.VMEM((NG, F), _f32),
        ],
    )(s3, g3, dinv_col, b3.reshape(1, F), bp,
      CW1, Cb1.reshape(1, F // 2), CW2, Cb2.reshape(1, F // 4),
      CW3, Cb3.reshape(1, 4))
    return z
